# Initial kernel scaffold; baseline (speedup 1.0000x reference)
import jax
import jax.numpy as jnp
from jax.experimental import pallas as pl


def kernel(center_nodes, context_nodes, negative_nodes, W_center, W_context):
    raise NotImplementedError("write your pallas kernel here")


# speedup vs baseline: 1.0609x; 1.0609x over previous
import functools

import jax
import jax.numpy as jnp
from jax import lax
from jax.experimental import pallas as pl
from jax.experimental.pallas import tpu as pltpu
from jax.experimental.pallas import tpu_sc as plsc

B = 16384
D = 32
K = 20
NW = 32
BPW = B // NW
C = 64
NCHUNK = BPW // C
GROUPS = C // 16
NEG_IDX_ROWS = C * K // 128

_LN2 = 0.6931471805599453


def _plog(x):
    bits = lax.bitcast_convert_type(x, jnp.int32)
    e = lax.shift_right_arithmetic(bits, 23) - 127
    m = lax.bitcast_convert_type(
        lax.bitwise_or(lax.bitwise_and(bits, 0x007FFFFF), 0x3F800000),
        jnp.float32)
    big = m > 1.4142135623730951
    m = jnp.where(big, m * 0.5, m)
    ef = e.astype(jnp.float32) + jnp.where(big, 1.0, 0.0)
    t = (m - 1.0) / (m + 1.0)
    t2 = t * t
    p = 1.0 + t2 * (0.33333333333 + t2 * (0.2 + t2 * (0.14285714285 + t2 * 0.11111111111)))
    return ef * _LN2 + 2.0 * t * p


def _loss(t):
    sig = 1.0 / (1.0 + jnp.exp(-t))
    return -_plog(sig + 1e-9)


_GATHER_DNUMS = lax.GatherDimensionNumbers(
    offset_dims=(), collapsed_slice_dims=(0,), start_index_map=(0,))


def _permute(x, idx2d):
    return lax.gather(x, idx2d, _GATHER_DNUMS, (1,),
                      mode=lax.GatherScatterMode.PROMISE_IN_BOUNDS)


def _tree_reduce16(ps, perms, sel):
    level = 0
    while len(ps) > 1:
        idx, msk = perms[level], sel[level]
        ps = [jnp.where(msk, a + _permute(a, idx), b + _permute(b, idx))
              for a, b in zip(ps[0::2], ps[1::2])]
        level += 1
    return ps[0]


def _skipgram_partials(cen_idx, ctx_idx, neg_idx2d, w_center, w_context):
    mesh = plsc.VectorSubcoreMesh(core_axis_name="c", subcore_axis_name="s")

    @functools.partial(
        pl.kernel,
        out_type=jax.ShapeDtypeStruct((NW, 16), jnp.float32),
        mesh=mesh,
        compiler_params=pltpu.CompilerParams(use_tc_tiling_on_sc=False),
        scratch_types=[
            pltpu.VMEM((C,), jnp.int32),
            pltpu.VMEM((C,), jnp.int32),
            pltpu.VMEM((C * K,), jnp.int32),
            pltpu.VMEM((C, D), jnp.float32),
            pltpu.VMEM((C, D), jnp.float32),
            pltpu.VMEM((C * K, D), jnp.float32),
            pltpu.VMEM((16,), jnp.float32),
            pltpu.SemaphoreType.DMA,
        ],
    )
    def body(cen_hbm, ctx_hbm, neg_hbm, wc_hbm, wx_hbm, out_hbm,
             cenidx_v, ctxidx_v, negidx_v, cen_v, ctx_v, neg_v, accv, sem):
        wid = lax.axis_index("s") * 2 + lax.axis_index("c")
        lane = lax.iota(jnp.int32, 16)
        perms = [(lane ^ s).reshape(16, 1) for s in (8, 4, 2, 1)]
        sel = [(lane & s) == 0 for s in (8, 4, 2, 1)]
        base = wid * BPW

        def chunk_body(c, acc):
            cb = base + c * C
            pltpu.sync_copy(cen_hbm.at[pl.ds(cb, C)], cenidx_v)
            pltpu.sync_copy(ctx_hbm.at[pl.ds(cb, C)], ctxidx_v)
            pltpu.sync_copy(neg_hbm.at[pl.ds(cb * K, C * K)], negidx_v)
            cps = [pltpu.async_copy(wc_hbm.at[cenidx_v], cen_v, sem),
                   pltpu.async_copy(wx_hbm.at[ctxidx_v], ctx_v, sem)]
            for j in range(NEG_IDX_ROWS):
                cps.append(pltpu.async_copy(
                    wx_hbm.at[negidx_v.at[pl.ds(j * 128, 128)]],
                    neg_v.at[pl.ds(j * 128, 128)], sem))
            for cp in cps:
                cp.wait()

            def group_body(g, acc2):
                eb = g * 16
                st = {"pend": [], "acc": acc2}

                def push(p):
                    st["pend"].append(p)
                    if len(st["pend"]) == 16:
                        st["acc"] = st["acc"] + _loss(
                            _tree_reduce16(st["pend"], perms, sel))
                        st["pend"] = []

                for i in range(16):
                    e = eb + i
                    c0 = cen_v[e, 0:16]
                    c1 = cen_v[e, 16:32]
                    x0 = ctx_v[e, 0:16]
                    x1 = ctx_v[e, 16:32]
                    push(c0 * x0 + c1 * x1)
                    nc0 = -c0
                    nc1 = -c1
                    for k in range(K):
                        r = e * K + k
                        push(neg_v[r, 0:16] * nc0 + neg_v[r, 16:32] * nc1)
                return st["acc"]

            return lax.fori_loop(0, GROUPS, group_body, acc)

        acc = lax.fori_loop(0, NCHUNK, chunk_body, jnp.zeros((16,), jnp.float32))
        accv[...] = acc
        pltpu.sync_copy(accv, out_hbm.at[wid])

    return body(cen_idx, ctx_idx, neg_idx2d, w_center, w_context)


def kernel(center_nodes, context_nodes, negative_nodes, W_center, W_context):
    cen = center_nodes.astype(jnp.int32)
    ctx = context_nodes.astype(jnp.int32)
    neg = negative_nodes.astype(jnp.int32).reshape(B * K)
    parts = _skipgram_partials(cen, ctx, neg, W_center, W_context)
    return jnp.sum(parts) * (1.0 / B)
